# baseline (speedup 1.0000x reference)
import functools

import jax
import jax.numpy as jnp
from jax import lax
from jax.experimental import pallas as pl
from jax.experimental.pallas import tpu as pltpu
from jax.experimental.pallas import tpu_sc as plsc

VOCAB = 1000000
D = 64
DH = 32
DHP = 64
DP = 128
B = 4096
L = 200
BF = B * L

NC = 2
NS = 16
NW = NC * NS
BPW = B // NW

NCHUNK = BPW

_mesh = plsc.VectorSubcoreMesh(core_axis_name="c", subcore_axis_name="s")


@functools.partial(
    pl.kernel,
    mesh=_mesh,
    out_type=jax.ShapeDtypeStruct((BF, DP), jnp.float32),
    scratch_types=[
        pltpu.VMEM((BPW, L), jnp.int32),
        pltpu.VMEM((L, DHP), jnp.float32),
        pltpu.VMEM((L, DHP), jnp.float32),
        pltpu.VMEM((L, DHP), jnp.float32),
        pltpu.VMEM((L, DHP), jnp.float32),
        pltpu.SemaphoreType.DMA,
        pltpu.SemaphoreType.DMA,
        pltpu.SemaphoreType.DMA,
        pltpu.SemaphoreType.DMA,
    ],
    compiler_params=pltpu.CompilerParams(use_tc_tiling_on_sc=False),
)
def _embed_sc(x_hbm, tl_hbm, tr_hbm, out_hbm, idx_v, rl0, rl1, rr0, rr1,
              gsem0, gsem1, ssem0, ssem1):
    wid = lax.axis_index("s") * NC + lax.axis_index("c")
    base = wid * BPW

    pltpu.sync_copy(x_hbm.at[pl.ds(base, BPW)], idx_v)

    rl = (rl0, rl1)
    rr = (rr0, rr1)
    gsem = (gsem0, gsem1)
    ssem = (ssem0, ssem1)

    def fire_gathers(cc, b):
        pltpu.async_copy(tl_hbm.at[idx_v.at[cc]], rl[b], gsem[b])
        pltpu.async_copy(tr_hbm.at[idx_v.at[cc]], rr[b], gsem[b])

    def wait_gathers(cc, b):
        pltpu.make_async_copy(tl_hbm.at[idx_v.at[cc]], rl[b], gsem[b]).wait()
        pltpu.make_async_copy(tr_hbm.at[idx_v.at[cc]], rr[b], gsem[b]).wait()

    def out_l(cc):
        return out_hbm.at[pl.ds((base + cc) * L, L), pl.ds(0, DH)]

    def out_r(cc):
        return out_hbm.at[pl.ds((base + cc) * L, L), pl.ds(DH, DH)]

    def fire_scatters(cc, b):
        pltpu.async_copy(rl[b].at[:, pl.ds(0, DH)], out_l(cc), ssem[b])
        pltpu.async_copy(rr[b].at[:, pl.ds(0, DH)], out_r(cc), ssem[b])

    def wait_scatters(cc, b):
        pltpu.make_async_copy(rl[b].at[:, pl.ds(0, DH)], out_l(cc),
                              ssem[b]).wait()
        pltpu.make_async_copy(rr[b].at[:, pl.ds(0, DH)], out_r(cc),
                              ssem[b]).wait()

    fire_gathers(0, 0)

    def chunk_body(cc, b):
        @pl.when(cc >= 1)
        def _():
            wait_scatters(cc - 1, 1 - b)

        @pl.when(cc + 1 < NCHUNK)
        def _():
            fire_gathers(cc + 1, 1 - b)

        wait_gathers(cc, b)
        fire_scatters(cc, b)

    def outer(i, carry):
        cc = i * 2
        chunk_body(cc, 0)
        chunk_body(cc + 1, 1)
        return carry

    lax.fori_loop(0, NCHUNK // 2, outer, 0)

    wait_scatters(NCHUNK - 1, 1)


def kernel(x, table):
    tl = jnp.pad(table[:, :DH], ((0, 0), (0, DHP - DH)))
    tr = jnp.pad(table[:, DH:], ((0, 0), (0, DHP - DH)))
    outp = _embed_sc(x.astype(jnp.int32), tl, tr)
    return outp.reshape(B, L, DP)[:, :, :D]


# speedup vs baseline: 2.5502x; 2.5502x over previous
import functools

import jax
import jax.numpy as jnp
from jax import lax
from jax.experimental import pallas as pl
from jax.experimental.pallas import tpu as pltpu
from jax.experimental.pallas import tpu_sc as plsc

VOCAB = 1000000
D = 64
DP = 128
B = 4096
L = 200
BF = B * L

NC = 2
NS = 16
NW = NC * NS
BPW = B // NW

NCHUNK = BPW

_mesh = plsc.VectorSubcoreMesh(core_axis_name="c", subcore_axis_name="s")


@functools.partial(
    pl.kernel,
    mesh=_mesh,
    out_type=jax.ShapeDtypeStruct((BF, DP), jnp.float32),
    scratch_types=[
        pltpu.VMEM((BPW, L), jnp.int32),
        pltpu.VMEM((L, DP), jnp.float32),
        pltpu.VMEM((L, DP), jnp.float32),
        pltpu.VMEM((L, DP), jnp.float32),
        pltpu.SemaphoreType.DMA,
        pltpu.SemaphoreType.DMA,
        pltpu.SemaphoreType.DMA,
        pltpu.SemaphoreType.DMA,
        pltpu.SemaphoreType.DMA,
        pltpu.SemaphoreType.DMA,
    ],
    compiler_params=pltpu.CompilerParams(use_tc_tiling_on_sc=False),
)
def _embed_sc(x_hbm, table_hbm, out_hbm, idx_v, rows0, rows1, rows2,
              gsem0, gsem1, gsem2, ssem0, ssem1, ssem2):
    wid = lax.axis_index("s") * NC + lax.axis_index("c")
    base = wid * BPW

    pltpu.sync_copy(x_hbm.at[pl.ds(base, BPW)], idx_v)

    rows = (rows0, rows1, rows2)
    gsem = (gsem0, gsem1, gsem2)
    ssem = (ssem0, ssem1, ssem2)

    def fire_gather(cc, b):
        pltpu.async_copy(table_hbm.at[idx_v.at[cc]], rows[b], gsem[b])

    def wait_gather(cc, b):
        pltpu.make_async_copy(table_hbm.at[idx_v.at[cc]], rows[b],
                              gsem[b]).wait()

    def out_ref(cc):
        return out_hbm.at[pl.ds((base + cc) * L, L), pl.ds(0, D)]

    def fire_scatter(cc, b):
        pltpu.async_copy(rows[b].at[:, pl.ds(0, D)], out_ref(cc), ssem[b])

    def wait_scatter(cc, b):
        pltpu.make_async_copy(rows[b].at[:, pl.ds(0, D)], out_ref(cc),
                              ssem[b]).wait()

    fire_gather(0, 0)
    fire_gather(1, 1)

    def chunk_body(cc, b):
        @pl.when(cc >= 1)
        def _():
            wait_scatter(cc - 1, (b + 2) % 3)

        @pl.when(cc + 2 < NCHUNK)
        def _():
            fire_gather(cc + 2, (b + 2) % 3)

        wait_gather(cc, b)
        fire_scatter(cc, b)

    def outer(i, carry):
        cc = i * 3
        chunk_body(cc, 0)
        chunk_body(cc + 1, 1)
        chunk_body(cc + 2, 2)
        return carry

    lax.fori_loop(0, NCHUNK // 3, outer, 0)

    for cc in range(3 * (NCHUNK // 3), NCHUNK):
        b = cc % 3
        wait_scatter(cc - 1, (b + 2) % 3)
        wait_gather(cc, b)
        fire_scatter(cc, b)

    wait_scatter(NCHUNK - 1, (NCHUNK - 1) % 3)


def kernel(x, table):
    tp = jnp.pad(table, ((0, 0), (0, DP - D)))
    outp = _embed_sc(x.astype(jnp.int32), tp)
    return outp.reshape(B, L, DP)[:, :, :D]
